# baseline (speedup 1.0000x reference)
import functools

import jax
import jax.numpy as jnp
from jax import lax
from jax.experimental import pallas as pl
from jax.experimental.pallas import tpu as pltpu
from jax.experimental.pallas import tpu_sc as plsc

N_NODES = 10000
N_EDGES = 320000
D = 128

NC = 2
NS = 16
NW = NC * NS

CHUNK = 128
CH = 160
E_PAD = NS * CH * CHUNK
DH = D // NC
CH_DEG = E_PAD // (NW * CHUNK)
N_PAD = 10240
ROWS_PER_TILE = N_PAD // NS
DEG_W = 16

_mesh = plsc.VectorSubcoreMesh(core_axis_name="c", subcore_axis_name="s",
                               num_cores=NC, num_subcores=NS)


def _deg_body(dst_hbm, ones_hbm, zeros_hbm, deg_hbm,
              idx_v, ones_v, deg_sh, sem):
    cid = lax.axis_index("c")
    sid = lax.axis_index("s")
    wid = cid * NS + sid
    pltpu.sync_copy(dst_hbm.at[wid], idx_v)
    pltpu.sync_copy(ones_hbm, ones_v)
    r0 = sid * ROWS_PER_TILE
    pltpu.sync_copy(zeros_hbm.at[pl.ds(r0, ROWS_PER_TILE)],
                    deg_sh.at[pl.ds(r0, ROWS_PER_TILE)])
    plsc.subcore_barrier()

    def step(chunk, _):
        pltpu.sync_copy(ones_v, deg_sh.at[idx_v.at[chunk]], add=True)
        return 0

    lax.fori_loop(0, CH_DEG, step, 0)
    plsc.subcore_barrier()
    pltpu.sync_copy(deg_sh.at[pl.ds(r0, ROWS_PER_TILE)],
                    deg_hbm.at[cid].at[pl.ds(r0, ROWS_PER_TILE)])


NBUF = 8
LAG = 6
AC = 64
CHG = (E_PAD // NS) // AC


def _agg_body(hp_hbm, src_hbm, dst_hbm, zeros_hbm, out_hbm,
              src_v, dst_v, rows_bf, rows_f, agg_sh, *sems):
    gsem = list(sems)
    cid = lax.axis_index("c")
    sid = lax.axis_index("s")
    pltpu.sync_copy(src_hbm.at[sid], src_v)
    pltpu.sync_copy(dst_hbm.at[sid], dst_v)
    r0 = sid * ROWS_PER_TILE
    pltpu.sync_copy(zeros_hbm.at[pl.ds(r0, ROWS_PER_TILE)],
                    agg_sh.at[pl.ds(r0, ROWS_PER_TILE)])
    plsc.subcore_barrier()

    hp_c = hp_hbm.at[cid]

    def gather_start(c, b):
        pltpu.async_copy(hp_c.at[src_v.at[c]], rows_bf.at[b], gsem[b])

    def gather_wait(c, b):
        pltpu.make_async_copy(hp_c.at[src_v.at[c]], rows_bf.at[b],
                              gsem[b]).wait()

    def convert(b):
        rb = rows_bf.at[b]
        rf = rows_f.at[b]

        def crow(r, _):
            for g in range(2):
                w = rb[r, pl.ds(16 * g, 16)]
                lo = lax.bitcast_convert_type(w << 16, jnp.float32)
                hi = lax.bitcast_convert_type(w & jnp.int32(-65536),
                                              jnp.float32)
                rf[r, pl.ds(32 * g, 16)] = lo
                rf[r, pl.ds(32 * g + 16, 16)] = hi
            return 0

        lax.fori_loop(0, AC, crow, 0, unroll=2)

    def handle(j, bj):
        gather_wait(j, bj)
        convert(bj)
        pltpu.sync_copy(rows_f.at[bj], agg_sh.at[dst_v.at[j]], add=True)

    def step(grp, _):
        for b in range(NBUF):
            v = NBUF * grp + b
            bj = (b - LAG) % NBUF

            @pl.when(v >= LAG)
            def _():
                handle(v - LAG, bj)

            gather_start(v, b)
        return 0

    lax.fori_loop(0, CHG // NBUF, step, 0)
    for j in range(CHG - LAG, CHG):
        handle(j, j % NBUF)
    plsc.subcore_barrier()
    pltpu.sync_copy(agg_sh.at[pl.ds(r0, ROWS_PER_TILE)],
                    out_hbm.at[cid].at[pl.ds(r0, ROWS_PER_TILE)])


def _make_deg_kernel(interpret=False):
    return pl.kernel(
        _deg_body,
        out_type=jax.ShapeDtypeStruct((NC, N_PAD, DEG_W), jnp.float32),
        mesh=_mesh,
        scratch_types=[
            pltpu.VMEM((CH_DEG, CHUNK), jnp.int32),
            pltpu.VMEM((CHUNK, DEG_W), jnp.float32),
            pltpu.VMEM_SHARED((N_PAD, DEG_W), jnp.float32),
            pltpu.SemaphoreType.DMA,
        ],
        compiler_params=pltpu.CompilerParams(use_tc_tiling_on_sc=False),
        interpret=interpret,
    )


def _make_agg_kernel(interpret=False):
    return pl.kernel(
        _agg_body,
        out_type=jax.ShapeDtypeStruct((NC, N_PAD, DH), jnp.float32),
        mesh=_mesh,
        scratch_types=[
            pltpu.VMEM((CHG, AC), jnp.int32),
            pltpu.VMEM((CHG, AC), jnp.int32),
            pltpu.VMEM((NBUF, AC, DH // 2), jnp.int32),
            pltpu.VMEM((NBUF, AC, DH), jnp.float32),
            pltpu.VMEM_SHARED((N_PAD, DH), jnp.float32),
        ] + [pltpu.SemaphoreType.DMA] * NBUF,
        compiler_params=pltpu.CompilerParams(use_tc_tiling_on_sc=False),
        interpret=interpret,
    )


_deg_kernel = _make_deg_kernel()
_agg_kernel = _make_agg_kernel()


def _hprime_body(x_ref, wct_ref, degp_ref, hp_ref):
    deg = degp_ref[0, :, 0:1] + degp_ref[1, :, 0:1] + 1.0
    dis = lax.rsqrt(deg)
    h = jnp.dot(x_ref[...], wct_ref[...], preferred_element_type=jnp.float32)
    hp = h * dis
    hp_ref[0] = hp[:, :DH]
    hp_ref[1] = hp[:, DH:]


def _head_body(p_ref, hp_ref, degp_ref, wht_ref, bc_ref, bh_ref, out_ref):
    deg = degp_ref[0, :, 0:1] + degp_ref[1, :, 0:1] + 1.0
    dis = lax.rsqrt(deg)
    agg = jnp.concatenate(
        [p_ref[0] + hp_ref[0], p_ref[1] + hp_ref[1]], axis=1)
    t = dis * agg + bc_ref[...]
    t = jnp.maximum(t, 0.0)
    out_ref[...] = (
        jnp.dot(t, wht_ref[...], preferred_element_type=jnp.float32)
        + bh_ref[...]
    )


_BLK = 512
_GRID = N_PAD // _BLK


def _row_spec():
    return pl.BlockSpec((_BLK, D), lambda i: (i, 0))


def _degp_spec():
    return pl.BlockSpec((NC, _BLK, DEG_W), lambda i: (0, i, 0))


def _full_spec(shape):
    return pl.BlockSpec(shape, lambda i: tuple(0 for _ in shape))


@jax.jit
def kernel(x, edge_index, W_conv, b_conv, W_head, b_head):
    ei = edge_index.astype(jnp.int32)
    pad = jnp.full((E_PAD - N_EDGES,), N_NODES, dtype=jnp.int32)
    src_flat = jnp.concatenate([ei[0], pad])
    dst_flat = jnp.concatenate([ei[1], pad])
    src = src_flat.reshape(NS, CHG, AC)
    dst = dst_flat.reshape(NS, CHG, AC)
    dst_deg = dst_flat.reshape(NW, CH_DEG, CHUNK)

    ones_deg = jnp.ones((CHUNK, DEG_W), jnp.float32)
    zeros_deg = jnp.zeros((N_PAD, DEG_W), jnp.float32)
    deg_p = _deg_kernel(dst_deg, ones_deg, zeros_deg)

    x_pad = jnp.zeros((N_PAD, D), x.dtype).at[:N_NODES].set(x)
    split_spec = pl.BlockSpec((NC, _BLK, DH), lambda i: (0, i, 0))
    hp = pl.pallas_call(
        _hprime_body,
        grid=(_GRID,),
        in_specs=[_row_spec(), _full_spec((D, D)), _degp_spec()],
        out_specs=split_spec,
        out_shape=jax.ShapeDtypeStruct((NC, N_PAD, DH), jnp.float32),
    )(x_pad, W_conv.T, deg_p)

    hu = jax.lax.bitcast_convert_type(hp.astype(jnp.bfloat16), jnp.uint16)
    hu = hu.reshape(NC, N_PAD, 2, 2, 16).astype(jnp.uint32)
    w = hu[:, :, :, 0, :] | (hu[:, :, :, 1, :] << 16)
    hpb = jax.lax.bitcast_convert_type(
        w.reshape(NC, N_PAD, DH // 2), jnp.int32)
    zeros_agg = jnp.zeros((N_PAD, DH), jnp.float32)
    partials = _agg_kernel(hpb, src, dst, zeros_agg)

    out = pl.pallas_call(
        _head_body,
        grid=(_GRID,),
        in_specs=[
            split_spec,
            split_spec,
            _degp_spec(),
            _full_spec((D, D)),
            _full_spec((1, D)),
            _full_spec((1, D)),
        ],
        out_specs=_row_spec(),
        out_shape=jax.ShapeDtypeStruct((N_PAD, D), jnp.float32),
    )(partials, hp, deg_p, W_head.T, b_conv.reshape(1, D),
      b_head.reshape(1, D))
    return out[:N_NODES]


# speedup vs baseline: 1.0324x; 1.0324x over previous
import functools

import jax
import jax.numpy as jnp
from jax import lax
from jax.experimental import pallas as pl
from jax.experimental.pallas import tpu as pltpu
from jax.experimental.pallas import tpu_sc as plsc

N_NODES = 10000
N_EDGES = 320000
D = 128

NC = 2
NS = 16
NW = NC * NS

CHUNK = 128
CH = 160
E_PAD = NS * CH * CHUNK
DH = D // NC
CH_DEG = E_PAD // (NW * CHUNK)
N_PAD = 10240
ROWS_PER_TILE = N_PAD // NS
DEG_W = 16

_mesh = plsc.VectorSubcoreMesh(core_axis_name="c", subcore_axis_name="s",
                               num_cores=NC, num_subcores=NS)


def _deg_body(dst_hbm, ones_hbm, zeros_hbm, deg_hbm,
              idx_v, ones_v, deg_sh, sem):
    cid = lax.axis_index("c")
    sid = lax.axis_index("s")
    wid = cid * NS + sid
    pltpu.sync_copy(dst_hbm.at[wid], idx_v)
    pltpu.sync_copy(ones_hbm, ones_v)
    r0 = sid * ROWS_PER_TILE
    pltpu.sync_copy(zeros_hbm.at[pl.ds(r0, ROWS_PER_TILE)],
                    deg_sh.at[pl.ds(r0, ROWS_PER_TILE)])
    plsc.subcore_barrier()

    def step(chunk, _):
        pltpu.sync_copy(ones_v, deg_sh.at[idx_v.at[chunk]], add=True)
        return 0

    lax.fori_loop(0, CH_DEG, step, 0)
    plsc.subcore_barrier()
    pltpu.sync_copy(deg_sh.at[pl.ds(r0, ROWS_PER_TILE)],
                    deg_hbm.at[cid].at[pl.ds(r0, ROWS_PER_TILE)])


NBUF = 8
LAG = 6
AC = 64
CHG = (E_PAD // NS) // AC


def _agg_body(hp_hbm, src_hbm, dst_hbm, zeros_hbm, out_hbm,
              src_v, dst_v, rows_bf, rows_f, agg_sh, *sems):
    gsem = list(sems)
    cid = lax.axis_index("c")
    sid = lax.axis_index("s")
    pltpu.sync_copy(src_hbm.at[sid], src_v)
    pltpu.sync_copy(dst_hbm.at[sid], dst_v)
    r0 = sid * ROWS_PER_TILE
    pltpu.sync_copy(zeros_hbm.at[pl.ds(r0, ROWS_PER_TILE)],
                    agg_sh.at[pl.ds(r0, ROWS_PER_TILE)])
    plsc.subcore_barrier()

    hp_c = hp_hbm.at[cid]

    def gather_start(c, b):
        pltpu.async_copy(hp_c.at[src_v.at[c]], rows_bf.at[b], gsem[b])

    def gather_wait(c, b):
        pltpu.make_async_copy(hp_c.at[src_v.at[c]], rows_bf.at[b],
                              gsem[b]).wait()

    def convert(b):
        rb = rows_bf.at[b]
        rf = rows_f.at[b]

        def crow(r, _):
            for g in range(2):
                w = rb[r, pl.ds(16 * g, 16)]
                lo = lax.bitcast_convert_type(w << 16, jnp.float32)
                hi = lax.bitcast_convert_type(w, jnp.float32)
                rf[r, pl.ds(32 * g, 16)] = lo
                rf[r, pl.ds(32 * g + 16, 16)] = hi
            return 0

        lax.fori_loop(0, AC, crow, 0, unroll=8)

    def handle(j, bj):
        gather_wait(j, bj)
        convert(bj)
        pltpu.sync_copy(rows_f.at[bj], agg_sh.at[dst_v.at[j]], add=True)

    def step(grp, _):
        for b in range(NBUF):
            v = NBUF * grp + b
            bj = (b - LAG) % NBUF

            @pl.when(v >= LAG)
            def _():
                handle(v - LAG, bj)

            gather_start(v, b)
        return 0

    lax.fori_loop(0, CHG // NBUF, step, 0)
    for j in range(CHG - LAG, CHG):
        handle(j, j % NBUF)
    plsc.subcore_barrier()
    pltpu.sync_copy(agg_sh.at[pl.ds(r0, ROWS_PER_TILE)],
                    out_hbm.at[cid].at[pl.ds(r0, ROWS_PER_TILE)])


def _make_deg_kernel(interpret=False):
    return pl.kernel(
        _deg_body,
        out_type=jax.ShapeDtypeStruct((NC, N_PAD, DEG_W), jnp.float32),
        mesh=_mesh,
        scratch_types=[
            pltpu.VMEM((CH_DEG, CHUNK), jnp.int32),
            pltpu.VMEM((CHUNK, DEG_W), jnp.float32),
            pltpu.VMEM_SHARED((N_PAD, DEG_W), jnp.float32),
            pltpu.SemaphoreType.DMA,
        ],
        compiler_params=pltpu.CompilerParams(use_tc_tiling_on_sc=False),
        interpret=interpret,
    )


def _make_agg_kernel(interpret=False):
    return pl.kernel(
        _agg_body,
        out_type=jax.ShapeDtypeStruct((NC, N_PAD, DH), jnp.float32),
        mesh=_mesh,
        scratch_types=[
            pltpu.VMEM((CHG, AC), jnp.int32),
            pltpu.VMEM((CHG, AC), jnp.int32),
            pltpu.VMEM((NBUF, AC, DH // 2), jnp.int32),
            pltpu.VMEM((NBUF, AC, DH), jnp.float32),
            pltpu.VMEM_SHARED((N_PAD, DH), jnp.float32),
        ] + [pltpu.SemaphoreType.DMA] * NBUF,
        compiler_params=pltpu.CompilerParams(use_tc_tiling_on_sc=False),
        interpret=interpret,
    )


_deg_kernel = _make_deg_kernel()
_agg_kernel = _make_agg_kernel()


def _hprime_body(x_ref, wct_ref, degp_ref, hp_ref):
    deg = degp_ref[0, :, 0:1] + degp_ref[1, :, 0:1] + 1.0
    dis = lax.rsqrt(deg)
    h = jnp.dot(x_ref[...], wct_ref[...], preferred_element_type=jnp.float32)
    hp = h * dis
    hp_ref[0] = hp[:, :DH]
    hp_ref[1] = hp[:, DH:]


def _head_body(p_ref, hp_ref, degp_ref, wht_ref, bc_ref, bh_ref, out_ref):
    deg = degp_ref[0, :, 0:1] + degp_ref[1, :, 0:1] + 1.0
    dis = lax.rsqrt(deg)
    agg = jnp.concatenate(
        [p_ref[0] + hp_ref[0], p_ref[1] + hp_ref[1]], axis=1)
    t = dis * agg + bc_ref[...]
    t = jnp.maximum(t, 0.0)
    out_ref[...] = (
        jnp.dot(t, wht_ref[...], preferred_element_type=jnp.float32)
        + bh_ref[...]
    )


_BLK = 512
_GRID = N_PAD // _BLK


def _row_spec():
    return pl.BlockSpec((_BLK, D), lambda i: (i, 0))


def _degp_spec():
    return pl.BlockSpec((NC, _BLK, DEG_W), lambda i: (0, i, 0))


def _full_spec(shape):
    return pl.BlockSpec(shape, lambda i: tuple(0 for _ in shape))


@jax.jit
def kernel(x, edge_index, W_conv, b_conv, W_head, b_head):
    ei = edge_index.astype(jnp.int32)
    pad = jnp.full((E_PAD - N_EDGES,), N_NODES, dtype=jnp.int32)
    src_flat = jnp.concatenate([ei[0], pad])
    dst_flat = jnp.concatenate([ei[1], pad])
    src = src_flat.reshape(NS, CHG, AC)
    dst = dst_flat.reshape(NS, CHG, AC)
    dst_deg = dst_flat.reshape(NW, CH_DEG, CHUNK)

    ones_deg = jnp.ones((CHUNK, DEG_W), jnp.float32)
    zeros_deg = jnp.zeros((N_PAD, DEG_W), jnp.float32)
    deg_p = _deg_kernel(dst_deg, ones_deg, zeros_deg)

    x_pad = jnp.zeros((N_PAD, D), x.dtype).at[:N_NODES].set(x)
    split_spec = pl.BlockSpec((NC, _BLK, DH), lambda i: (0, i, 0))
    hp = pl.pallas_call(
        _hprime_body,
        grid=(_GRID,),
        in_specs=[_row_spec(), _full_spec((D, D)), _degp_spec()],
        out_specs=split_spec,
        out_shape=jax.ShapeDtypeStruct((NC, N_PAD, DH), jnp.float32),
    )(x_pad, W_conv.T, deg_p)

    hu = jax.lax.bitcast_convert_type(hp.astype(jnp.bfloat16), jnp.uint16)
    hu = hu.reshape(NC, N_PAD, 2, 2, 16).astype(jnp.uint32)
    w = hu[:, :, :, 0, :] | (hu[:, :, :, 1, :] << 16)
    hpb = jax.lax.bitcast_convert_type(
        w.reshape(NC, N_PAD, DH // 2), jnp.int32)
    zeros_agg = jnp.zeros((N_PAD, DH), jnp.float32)
    partials = _agg_kernel(hpb, src, dst, zeros_agg)

    out = pl.pallas_call(
        _head_body,
        grid=(_GRID,),
        in_specs=[
            split_spec,
            split_spec,
            _degp_spec(),
            _full_spec((D, D)),
            _full_spec((1, D)),
            _full_spec((1, D)),
        ],
        out_specs=_row_spec(),
        out_shape=jax.ShapeDtypeStruct((N_PAD, D), jnp.float32),
    )(partials, hp, deg_p, W_head.T, b_conv.reshape(1, D),
      b_head.reshape(1, D))
    return out[:N_NODES]
